# baseline (speedup 1.0000x reference)
import functools

import jax
import jax.numpy as jnp
from jax import lax
from jax.experimental import pallas as pl
from jax.experimental.pallas import tpu as pltpu
from jax.experimental.pallas import tpu_sc as plsc

HIDDEN = 768
EPS = 1e-05
LANES = 16
NSL = HIDDEN // LANES
NC, NS = 2, 16
NW = NC * NS
CHUNK = 32


def _shuffle(x, idx):
    dnums = lax.GatherDimensionNumbers(
        offset_dims=(), collapsed_slice_dims=(0,), start_index_map=(0,)
    )
    return lax.gather(
        x,
        idx[:, None],
        dnums,
        (1,),
        mode=lax.GatherScatterMode.PROMISE_IN_BOUNDS,
    )


def _bcast_sum(*vs):
    vs = list(vs)
    lanes = lax.iota(jnp.int32, LANES)
    for sh in (1, 2, 4, 8):
        idx = jnp.bitwise_xor(lanes, jnp.int32(sh))
        vs = [v + _shuffle(v, idx) for v in vs]
    return vs


def _rsqrt_vec(v):
    i = lax.bitcast_convert_type(v, jnp.int32)
    i = jnp.int32(0x5F3759DF) - lax.shift_right_logical(i, 1)
    y = lax.bitcast_convert_type(i, jnp.float32)
    half = v * jnp.float32(0.5)
    for _ in range(3):
        y = y * (jnp.float32(1.5) - half * y * y)
    return y


def _sc_body(
    n_rows,
    ids_hbm,
    table_hbm,
    w_hbm,
    out_hbm,
    idx_v,
    rows0,
    rows1,
    out0,
    out1,
    w_v,
    gsem0,
    gsem1,
    ssem0,
    ssem1,
):
    wid = lax.axis_index("s") * NC + lax.axis_index("c")
    rows_per_w = n_rows // NW
    base = wid * rows_per_w
    pltpu.sync_copy(ids_hbm.at[pl.ds(base, rows_per_w)], idx_v)
    pltpu.sync_copy(w_hbm, w_v)

    n_chunks = rows_per_w // CHUNK

    def start_gather(g, buf, sem):
        pltpu.async_copy(
            table_hbm.at[idx_v.at[pl.ds(g * CHUNK, CHUNK)]],
            buf.at[pl.ds(0, CHUNK)],
            sem,
        )

    def wait_gather(g, buf, sem):
        pltpu.make_async_copy(
            table_hbm.at[idx_v.at[pl.ds(g * CHUNK, CHUNK)]],
            buf.at[pl.ds(0, CHUNK)],
            sem,
        ).wait()

    def start_scatter(g, obuf, sem):
        pltpu.async_copy(obuf, out_hbm.at[pl.ds(base + g * CHUNK, CHUNK)], sem)

    def wait_scatter(g, obuf, sem):
        pltpu.make_async_copy(
            obuf, out_hbm.at[pl.ds(base + g * CHUNK, CHUNK)], sem
        ).wait()

    def layer_norm(rows_v, out_v):
        zero = jnp.zeros((LANES,), jnp.float32)
        s0 = q0 = s1 = q1 = zero
        for j in range(NSL):
            sl = pl.ds(j * LANES, LANES)
            x0 = rows_v[0, sl]
            s0 = s0 + x0
            q0 = q0 + x0 * x0
            x1 = rows_v[1, sl]
            s1 = s1 + x1
            q1 = q1 + x1 * x1

        def finalize(s, q):
            mean_v = s * jnp.float32(1.0 / HIDDEN)
            var_v = q * jnp.float32(1.0 / HIDDEN) - mean_v * mean_v
            inv_v = _rsqrt_vec(var_v + jnp.float32(EPS))
            return inv_v, mean_v * inv_v

        def pair_body(i, carry):
            sa, qa, sb, qb = carry
            r = 2 * i
            st0, qt0, st1, qt1 = _bcast_sum(sa, qa, sb, qb)
            inv0, minv0 = finalize(st0, qt0)
            inv1, minv1 = finalize(st1, qt1)
            ns0 = nq0 = ns1 = nq1 = zero
            for j in range(NSL):
                sl = pl.ds(j * LANES, LANES)
                wv = w_v[sl]
                xn0 = rows_v[r + 2, sl]
                ns0 = ns0 + xn0
                nq0 = nq0 + xn0 * xn0
                xn1 = rows_v[r + 3, sl]
                ns1 = ns1 + xn1
                nq1 = nq1 + xn1 * xn1
                x0 = rows_v[r, sl]
                out_v[r, sl] = (x0 * inv0 - minv0) * wv
                x1 = rows_v[r + 1, sl]
                out_v[r + 1, sl] = (x1 * inv1 - minv1) * wv
            return ns0, nq0, ns1, nq1

        lax.fori_loop(0, CHUNK // 2, pair_body, (s0, q0, s1, q1))

    def process(g, buf, obuf, gsem, ssem):
        wait_gather(g, buf, gsem)

        @pl.when(g >= 2)
        def _():
            wait_scatter(g - 2, obuf, ssem)

        layer_norm(buf, obuf)
        start_scatter(g, obuf, ssem)

        @pl.when(g + 2 < n_chunks)
        def _():
            start_gather(g + 2, buf, gsem)

    start_gather(0, rows0, gsem0)
    start_gather(1, rows1, gsem1)

    def pair_body(i, carry):
        process(2 * i, rows0, out0, gsem0, ssem0)
        process(2 * i + 1, rows1, out1, gsem1, ssem1)
        return carry

    lax.fori_loop(0, n_chunks // 2, pair_body, 0)
    wait_scatter(n_chunks - 2, out0, ssem0)
    wait_scatter(n_chunks - 1, out1, ssem1)


def kernel(input_ids, table, norm_weight):
    b, s = input_ids.shape
    n_rows = b * s
    ids_flat = input_ids.reshape((n_rows,)).astype(jnp.int32)

    mesh = plsc.VectorSubcoreMesh(core_axis_name="c", subcore_axis_name="s")
    rows_per_w = n_rows // NW

    sc_fn = pl.kernel(
        functools.partial(_sc_body, n_rows),
        out_type=jax.ShapeDtypeStruct((n_rows, HIDDEN), jnp.float32),
        mesh=mesh,
        scratch_types=[
            pltpu.VMEM((rows_per_w,), jnp.int32),
            pltpu.VMEM((CHUNK + 2, HIDDEN), jnp.float32),
            pltpu.VMEM((CHUNK + 2, HIDDEN), jnp.float32),
            pltpu.VMEM((CHUNK, HIDDEN), jnp.float32),
            pltpu.VMEM((CHUNK, HIDDEN), jnp.float32),
            pltpu.VMEM((HIDDEN,), jnp.float32),
            pltpu.SemaphoreType.DMA,
            pltpu.SemaphoreType.DMA,
            pltpu.SemaphoreType.DMA,
            pltpu.SemaphoreType.DMA,
        ],
    )
    out = sc_fn(ids_flat, table, norm_weight)
    return out.reshape((b, s, HIDDEN))


# speedup vs baseline: 2.0150x; 2.0150x over previous
import functools

import jax
import jax.numpy as jnp
from jax import lax
from jax.experimental import pallas as pl
from jax.experimental.pallas import tpu as pltpu
from jax.experimental.pallas import tpu_sc as plsc

HIDDEN = 768
EPS = 1e-05
LANES = 16
NSL = HIDDEN // LANES
NC, NS = 2, 16
NW = NC * NS
CHUNK = 64


def _shuffle(x, idx):
    dnums = lax.GatherDimensionNumbers(
        offset_dims=(), collapsed_slice_dims=(0,), start_index_map=(0,)
    )
    return lax.gather(
        x,
        idx[:, None],
        dnums,
        (1,),
        mode=lax.GatherScatterMode.PROMISE_IN_BOUNDS,
    )


def _bcast_sum(*vs):
    vs = list(vs)
    lanes = lax.iota(jnp.int32, LANES)
    for sh in (1, 2, 4, 8):
        idx = jnp.bitwise_xor(lanes, jnp.int32(sh))
        vs = [v + _shuffle(v, idx) for v in vs]
    return vs


def _rsqrt_vec(v):
    i = lax.bitcast_convert_type(v, jnp.int32)
    i = jnp.int32(0x5F3759DF) - lax.shift_right_logical(i, 1)
    y = lax.bitcast_convert_type(i, jnp.float32)
    half = v * jnp.float32(0.5)
    for _ in range(3):
        y = y * (jnp.float32(1.5) - half * y * y)
    return y


def _sc_body(
    n_rows, ids_hbm, table_hbm, w_hbm, out_hbm, idx_v, rows0, rows1, w_v, gsem0, gsem1, ssem0, ssem1
):
    wid = lax.axis_index("s") * NC + lax.axis_index("c")
    rows_per_w = n_rows // NW
    base = wid * rows_per_w
    pltpu.sync_copy(ids_hbm.at[pl.ds(base, rows_per_w)], idx_v)
    pltpu.sync_copy(w_hbm, w_v)

    n_chunks = rows_per_w // CHUNK

    def start_gather(g, buf, sem):
        pltpu.async_copy(
            table_hbm.at[idx_v.at[pl.ds(g * CHUNK, CHUNK)]],
            buf.at[pl.ds(0, CHUNK)],
            sem,
        )

    def wait_gather(g, buf, sem):
        pltpu.make_async_copy(
            table_hbm.at[idx_v.at[pl.ds(g * CHUNK, CHUNK)]],
            buf.at[pl.ds(0, CHUNK)],
            sem,
        ).wait()

    def start_scatter(g, buf, sem):
        pltpu.async_copy(
            buf.at[pl.ds(0, CHUNK)], out_hbm.at[pl.ds(base + g * CHUNK, CHUNK)], sem
        )

    def wait_scatter(g, buf, sem):
        pltpu.make_async_copy(
            buf.at[pl.ds(0, CHUNK)], out_hbm.at[pl.ds(base + g * CHUNK, CHUNK)], sem
        ).wait()

    def finalize(s, q):
        mean_v = s * jnp.float32(1.0 / HIDDEN)
        var_v = q * jnp.float32(1.0 / HIDDEN) - mean_v * mean_v
        inv_v = _rsqrt_vec(var_v + jnp.float32(EPS))
        return inv_v, mean_v * inv_v

    zero = jnp.zeros((LANES,), jnp.float32)

    def head_stats(rows_v):
        s0 = q0 = s1 = q1 = zero
        for j in range(NSL):
            sl = pl.ds(j * LANES, LANES)
            x0 = rows_v[0, sl]
            s0 = s0 + x0
            q0 = q0 + x0 * x0
            x1 = rows_v[1, sl]
            s1 = s1 + x1
            q1 = q1 + x1 * x1
        return s0, q0, s1, q1

    def norm_rows(rows_v, lo, hi, carry):
        def pair_body(i, carry):
            sa, qa, sb, qb = carry
            r = 2 * i
            st0, qt0, st1, qt1 = _bcast_sum(sa, qa, sb, qb)
            inv0, minv0 = finalize(st0, qt0)
            inv1, minv1 = finalize(st1, qt1)
            ns0 = nq0 = ns1 = nq1 = zero
            for j in range(NSL):
                sl = pl.ds(j * LANES, LANES)
                wv = w_v[sl]
                xn0 = rows_v[r + 2, sl]
                ns0 = ns0 + xn0
                nq0 = nq0 + xn0 * xn0
                xn1 = rows_v[r + 3, sl]
                ns1 = ns1 + xn1
                nq1 = nq1 + xn1 * xn1
                x0 = rows_v[r, sl]
                rows_v[r, sl] = (x0 * inv0 - minv0) * wv
                x1 = rows_v[r + 1, sl]
                rows_v[r + 1, sl] = (x1 * inv1 - minv1) * wv
            return ns0, nq0, ns1, nq1

        return lax.fori_loop(lo // 2, hi // 2, pair_body, carry)

    def process(g, buf, other_buf, gsem, other_gsem, ssem, other_ssem):
        wait_gather(g, buf, gsem)
        carry = head_stats(buf)
        carry = norm_rows(buf, 0, CHUNK // 2, carry)

        @pl.when(g >= 1)
        def _():
            wait_scatter(g - 1, other_buf, other_ssem)

        @pl.when(g + 1 < n_chunks)
        def _():
            start_gather(g + 1, other_buf, other_gsem)

        norm_rows(buf, CHUNK // 2, CHUNK, carry)
        start_scatter(g, buf, ssem)

    def pair_body(i, carry):
        g0 = 2 * i
        process(g0, rows0, rows1, gsem0, gsem1, ssem0, ssem1)
        process(g0 + 1, rows1, rows0, gsem1, gsem0, ssem1, ssem0)
        return carry

    start_gather(0, rows0, gsem0)
    lax.fori_loop(0, n_chunks // 2, pair_body, 0)
    wait_scatter(n_chunks - 1, rows1, ssem1)


def kernel(input_ids, table, norm_weight):
    b, s = input_ids.shape
    n_rows = b * s
    ids_flat = input_ids.reshape((n_rows,)).astype(jnp.int32)

    mesh = plsc.VectorSubcoreMesh(core_axis_name="c", subcore_axis_name="s")
    rows_per_w = n_rows // NW

    sc_fn = pl.kernel(
        functools.partial(_sc_body, n_rows),
        out_type=jax.ShapeDtypeStruct((n_rows, HIDDEN), jnp.float32),
        mesh=mesh,
        scratch_types=[
            pltpu.VMEM((rows_per_w,), jnp.int32),
            pltpu.VMEM((CHUNK + 2, HIDDEN), jnp.float32),
            pltpu.VMEM((CHUNK + 2, HIDDEN), jnp.float32),
            pltpu.VMEM((HIDDEN,), jnp.float32),
            pltpu.SemaphoreType.DMA,
            pltpu.SemaphoreType.DMA,
            pltpu.SemaphoreType.DMA,
            pltpu.SemaphoreType.DMA,
        ],
    )
    out = sc_fn(ids_flat, table, norm_weight)
    return out.reshape((b, s, HIDDEN))


# speedup vs baseline: 2.0574x; 1.0210x over previous
import functools

import jax
import jax.numpy as jnp
from jax import lax
from jax.experimental import pallas as pl
from jax.experimental.pallas import tpu as pltpu
from jax.experimental.pallas import tpu_sc as plsc

HIDDEN = 768
EPS = 1e-05
LANES = 16
NSL = HIDDEN // LANES
NC, NS = 2, 16
NW = NC * NS
CHUNK = 64


def _shuffle(x, idx):
    dnums = lax.GatherDimensionNumbers(
        offset_dims=(), collapsed_slice_dims=(0,), start_index_map=(0,)
    )
    return lax.gather(
        x,
        idx[:, None],
        dnums,
        (1,),
        mode=lax.GatherScatterMode.PROMISE_IN_BOUNDS,
    )


def _bcast_sum(*vs):
    vs = list(vs)
    lanes = lax.iota(jnp.int32, LANES)
    for sh in (1, 2, 4, 8):
        idx = jnp.bitwise_xor(lanes, jnp.int32(sh))
        vs = [v + _shuffle(v, idx) for v in vs]
    return vs


def _rsqrt_vec(v):
    i = lax.bitcast_convert_type(v, jnp.int32)
    i = jnp.int32(0x5F3759DF) - lax.shift_right_logical(i, 1)
    y = lax.bitcast_convert_type(i, jnp.float32)
    half = v * jnp.float32(0.5)
    for _ in range(2):
        y = y * (jnp.float32(1.5) - half * y * y)
    return y


def _sc_body(
    n_rows, ids_hbm, table_hbm, w_hbm, out_hbm, idx_v, rows0, rows1, w_v, gsem0, gsem1, ssem0, ssem1
):
    wid = lax.axis_index("s") * NC + lax.axis_index("c")
    rows_per_w = n_rows // NW
    base = wid * rows_per_w
    pltpu.sync_copy(ids_hbm.at[pl.ds(base, rows_per_w)], idx_v)
    pltpu.sync_copy(w_hbm, w_v)

    n_chunks = rows_per_w // CHUNK

    def start_gather(g, buf, sem):
        pltpu.async_copy(
            table_hbm.at[idx_v.at[pl.ds(g * CHUNK, CHUNK)]],
            buf.at[pl.ds(0, CHUNK)],
            sem,
        )

    def wait_gather(g, buf, sem):
        pltpu.make_async_copy(
            table_hbm.at[idx_v.at[pl.ds(g * CHUNK, CHUNK)]],
            buf.at[pl.ds(0, CHUNK)],
            sem,
        ).wait()

    def start_scatter(g, buf, sem):
        pltpu.async_copy(
            buf.at[pl.ds(0, CHUNK)], out_hbm.at[pl.ds(base + g * CHUNK, CHUNK)], sem
        )

    def wait_scatter(g, buf, sem):
        pltpu.make_async_copy(
            buf.at[pl.ds(0, CHUNK)], out_hbm.at[pl.ds(base + g * CHUNK, CHUNK)], sem
        ).wait()

    def finalize(s, q):
        mean_v = s * jnp.float32(1.0 / HIDDEN)
        var_v = q * jnp.float32(1.0 / HIDDEN) - mean_v * mean_v
        inv_v = _rsqrt_vec(var_v + jnp.float32(EPS))
        return inv_v, mean_v * inv_v

    zero = jnp.zeros((LANES,), jnp.float32)

    GRP = 4

    def head_stats(rows_v):
        s = [zero] * GRP
        q = [zero] * GRP
        for j in range(NSL):
            sl = pl.ds(j * LANES, LANES)
            for k in range(GRP):
                x = rows_v[k, sl]
                s[k] = s[k] + x
                q[k] = q[k] + x * x
        return tuple(s) + tuple(q)

    def norm_rows(rows_v, lo, hi, carry):
        def group_body(i, carry):
            s, q = carry[:GRP], carry[GRP:]
            r = GRP * i
            tot = _bcast_sum(*s, *q)
            inv = [None] * GRP
            minv = [None] * GRP
            for k in range(GRP):
                inv[k], minv[k] = finalize(tot[k], tot[GRP + k])
            ns = [zero] * GRP
            nq = [zero] * GRP
            for j in range(NSL):
                sl = pl.ds(j * LANES, LANES)
                wv = w_v[sl]
                for k in range(GRP):
                    xn = rows_v[r + GRP + k, sl]
                    ns[k] = ns[k] + xn
                    nq[k] = nq[k] + xn * xn
                    x = rows_v[r + k, sl]
                    rows_v[r + k, sl] = (x * inv[k] - minv[k]) * wv
            return tuple(ns) + tuple(nq)

        return lax.fori_loop(lo // GRP, hi // GRP, group_body, carry)

    def process(g, buf, other_buf, gsem, other_gsem, ssem, other_ssem):
        wait_gather(g, buf, gsem)
        carry = head_stats(buf)
        carry = norm_rows(buf, 0, CHUNK // 2, carry)

        @pl.when(g >= 1)
        def _():
            wait_scatter(g - 1, other_buf, other_ssem)

        @pl.when(g + 1 < n_chunks)
        def _():
            start_gather(g + 1, other_buf, other_gsem)

        norm_rows(buf, CHUNK // 2, CHUNK, carry)
        start_scatter(g, buf, ssem)

    def pair_body(i, carry):
        g0 = 2 * i
        process(g0, rows0, rows1, gsem0, gsem1, ssem0, ssem1)
        process(g0 + 1, rows1, rows0, gsem1, gsem0, ssem1, ssem0)
        return carry

    start_gather(0, rows0, gsem0)
    lax.fori_loop(0, n_chunks // 2, pair_body, 0)
    wait_scatter(n_chunks - 1, rows1, ssem1)


def kernel(input_ids, table, norm_weight):
    b, s = input_ids.shape
    n_rows = b * s
    ids_flat = input_ids.reshape((n_rows,)).astype(jnp.int32)

    mesh = plsc.VectorSubcoreMesh(core_axis_name="c", subcore_axis_name="s")
    rows_per_w = n_rows // NW

    sc_fn = pl.kernel(
        functools.partial(_sc_body, n_rows),
        out_type=jax.ShapeDtypeStruct((n_rows, HIDDEN), jnp.float32),
        mesh=mesh,
        scratch_types=[
            pltpu.VMEM((rows_per_w,), jnp.int32),
            pltpu.VMEM((CHUNK + 4, HIDDEN), jnp.float32),
            pltpu.VMEM((CHUNK + 4, HIDDEN), jnp.float32),
            pltpu.VMEM((HIDDEN,), jnp.float32),
            pltpu.SemaphoreType.DMA,
            pltpu.SemaphoreType.DMA,
            pltpu.SemaphoreType.DMA,
            pltpu.SemaphoreType.DMA,
        ],
    )
    out = sc_fn(ids_flat, table, norm_weight)
    return out.reshape((b, s, HIDDEN))
